# baseline (speedup 1.0000x reference)
import jax
import jax.numpy as jnp
from jax import lax
from jax.experimental import pallas as pl
from jax.experimental.pallas import tpu as pltpu
from jax.experimental.pallas import tpu_sc as plsc

_EMB = 1024
_DFF = 2048
_E = 8
_W_IMPORTANCE = 0.01

_N = 4096
_A = 2 * _N
_TB = 256
_P = _A + _E * _TB
_NT = _P // _TB
_NW = 32
_TW = _N // _NW
_TM = 1024


def _gate_kernel(q_ref, wv_ref, bv_ref, wo_ref, bo_ref, gw_ref, gb_ref,
                 prob_ref, i1_ref, i2_ref, w1_ref, w2_ref, r1_ref, r2_ref,
                 imp_ref, counts_ref, eid_ref, carry_ref):
    t = pl.program_id(0)
    v = jnp.dot(q_ref[...], wv_ref[...], preferred_element_type=jnp.float32)
    v = v + bv_ref[...]
    att = jnp.dot(v, wo_ref[...], preferred_element_type=jnp.float32)
    att = att + bo_ref[...]
    logits = jnp.dot(att, gw_ref[...], preferred_element_type=jnp.float32)
    logits = logits + gb_ref[...]
    lmax = jnp.max(logits, axis=1, keepdims=True)
    ex = jnp.exp(logits - lmax)
    p = ex / jnp.sum(ex, axis=1, keepdims=True)
    prob_ref[...] = p

    iota = lax.broadcasted_iota(jnp.int32, p.shape, 1)
    m1 = jnp.max(p, axis=1, keepdims=True)
    i1 = jnp.min(jnp.where(p == m1, iota, _E), axis=1, keepdims=True)
    oh1 = iota == i1
    pm = jnp.where(oh1, -jnp.inf, p)
    m2 = jnp.max(pm, axis=1, keepdims=True)
    i2 = jnp.min(jnp.where(pm == m2, iota, _E), axis=1, keepdims=True)
    oh2 = iota == i2
    e21 = jnp.exp(m2 - m1)
    i1_ref[...] = i1
    i2_ref[...] = i2
    w1_ref[...] = 1.0 / (1.0 + e21)
    w2_ref[...] = e21 / (1.0 + e21)

    @pl.when(t == 0)
    def _():
        imp_ref[...] = jnp.zeros_like(imp_ref)
        carry_ref[...] = jnp.zeros_like(carry_ref)

    imp_ref[...] += jnp.sum(p, axis=0, keepdims=True)

    oh1f = oh1.astype(jnp.float32)
    oh2f = oh2.astype(jnp.float32)
    ohb = oh1f + oh2f
    tri = (lax.broadcasted_iota(jnp.int32, (_TM, _TM), 0) >=
           lax.broadcasted_iota(jnp.int32, (_TM, _TM), 1)).astype(jnp.float32)
    csum = jnp.dot(tri, ohb, preferred_element_type=jnp.float32)
    csum = csum + carry_ref[...]
    r1_ref[...] = (jnp.sum(oh1f * csum, axis=1, keepdims=True) - 1.0
                   ).astype(jnp.int32)
    r2_ref[...] = (jnp.sum(oh2f * csum, axis=1, keepdims=True) - 1.0
                   ).astype(jnp.int32)
    carry_ref[...] += jnp.sum(ohb, axis=0, keepdims=True)

    @pl.when(t == (_N // _TM) - 1)
    def _():
        cnt = carry_ref[...]
        counts_ref[...] = cnt.astype(jnp.int32)
        pc = jnp.floor((cnt + (_TB - 1)) / _TB) * _TB
        tri8 = (lax.broadcasted_iota(jnp.int32, (_E, _E), 0) <=
                lax.broadcasted_iota(jnp.int32, (_E, _E), 1)).astype(jnp.float32)
        cse = jnp.dot(pc, tri8, preferred_element_type=jnp.float32)
        ti = lax.broadcasted_iota(jnp.int32, (1, 48), 1) * _TB
        acc = jnp.zeros((1, 48), jnp.int32)
        for e in range(_E):
            acc += (ti >= cse[0, e].astype(jnp.int32)).astype(jnp.int32)
        eid_ref[...] = jnp.minimum(acc, _E - 1)


def _dispatch_kernel(i1_h, i2_h, r1_h, r2_h, counts_h, xb_h, w1_h, w2_h,
                     xs_h, ws_h, s1_h, s2_h,
                     cnt_v, base_v, iv_v, rv_v, wv_v, idx_v, xrow_v, sem):
    cid = lax.axis_index("c")
    sid = lax.axis_index("s")
    wid = sid * 2 + cid
    tokbase = wid * _TW

    pltpu.sync_copy(counts_h, cnt_v)
    c = cnt_v[...]
    pc = ((c + (_TB - 1)) >> 8) << 8
    cs = plsc.cumsum(pc)
    base_v[...] = cs - pc

    pltpu.sync_copy(i1_h.at[pl.ds(tokbase, _TW)], iv_v.at[0])
    pltpu.sync_copy(i2_h.at[pl.ds(tokbase, _TW)], iv_v.at[1])
    pltpu.sync_copy(r1_h.at[pl.ds(tokbase, _TW)], rv_v.at[0])
    pltpu.sync_copy(r2_h.at[pl.ds(tokbase, _TW)], rv_v.at[1])
    pltpu.sync_copy(w1_h.at[pl.ds(tokbase, _TW)], wv_v.at[0])
    pltpu.sync_copy(w2_h.at[pl.ds(tokbase, _TW)], wv_v.at[1])
    pltpu.sync_copy(xb_h.at[pl.ds(tokbase, _TW)], xrow_v)

    for k in range(2):
        for v in range(_TW // 16):
            e = iv_v[k, pl.ds(v * 16, 16)]
            r = rv_v[k, pl.ds(v * 16, 16)]
            b = plsc.load_gather(base_v, [e])
            idx_v[k, pl.ds(v * 16, 16)] = b + r

    copies = [
        pltpu.async_copy(xrow_v, xs_h.at[idx_v.at[0]], sem),
        pltpu.async_copy(xrow_v, xs_h.at[idx_v.at[1]], sem),
        pltpu.async_copy(wv_v.at[0], ws_h.at[idx_v.at[0]], sem),
        pltpu.async_copy(wv_v.at[1], ws_h.at[idx_v.at[1]], sem),
    ]
    for cp in copies:
        cp.wait()

    pltpu.sync_copy(idx_v.at[0], s1_h.at[wid])
    pltpu.sync_copy(idx_v.at[1], s2_h.at[wid])


def _expert_kernel(eid_ref, xs_ref, w1_ref, b1_ref, w2_ref, b2_ref, ws_ref,
                   ys_ref):
    h = jnp.dot(xs_ref[...], w1_ref[0], preferred_element_type=jnp.float32)
    h = jnp.maximum(h + b1_ref[0], 0.0).astype(jnp.bfloat16)
    part = jnp.dot(h, w2_ref[0], preferred_element_type=jnp.float32)
    ys_ref[...] = ((part + b2_ref[0]) * ws_ref[0]).astype(jnp.bfloat16)


def _combine_kernel(ys_h, s1_h, s2_h, y_h, sv_v, g1_v, g2_v, obuf_v,
                    sem0, sem1):
    cid = lax.axis_index("c")
    sid = lax.axis_index("s")
    wid = sid * 2 + cid
    tokbase = wid * _TW
    sems = (sem0, sem1)
    nv = _TW // 16
    hw = _EMB // 2

    pltpu.sync_copy(s1_h.at[wid], sv_v.at[0])
    pltpu.sync_copy(s2_h.at[wid], sv_v.at[1])

    def fire(v):
        par = v % 2
        return (
            pltpu.async_copy(
                ys_h.at[sv_v.at[0, pl.ds(v * 16, 16)]], g1_v.at[par],
                sems[par]),
            pltpu.async_copy(
                ys_h.at[sv_v.at[1, pl.ds(v * 16, 16)]], g2_v.at[par],
                sems[par]),
        )

    pend = fire(0)
    for v in range(nv):
        par = v % 2
        for cp in pend:
            cp.wait()
        if v + 1 < nv:
            pend = fire(v + 1)
        def row_body(r, _, par=par):
            def col_body(ci, _):
                for u in range(4):
                    off = ci * 64 + u * 16
                    a = plsc.bitcast(g1_v[par, r, pl.ds(off, 16)],
                                     jnp.bfloat16)
                    b = plsc.bitcast(g2_v[par, r, pl.ds(off, 16)],
                                     jnp.bfloat16)
                    obuf_v[r, pl.ds(off, 16)] = plsc.bitcast(
                        a + b, jnp.int32)
                return 0
            lax.fori_loop(0, hw // 64, col_body, 0)
            return 0
        lax.fori_loop(0, 16, row_body, 0)
        pltpu.sync_copy(obuf_v, y_h.at[pl.ds(tokbase + v * 16, 16)])


def kernel(x, q, Wq, bq, Wk, bk, Wv, bv, Wo, bo, gate_W, gate_b, W1, b1, W2, b2):
    x_shape = x.shape
    xf = x.reshape(-1, x_shape[-1])
    N, d = xf.shape
    T = N // _TM

    (gate_prob, i1, i2, w1, w2, r1, r2, imp, counts, eid_row
     ) = pl.pallas_call(
        _gate_kernel,
        grid=(T,),
        in_specs=[
            pl.BlockSpec((_TM, d), lambda t: (t, 0)),
            pl.BlockSpec((d, d), lambda t: (0, 0)),
            pl.BlockSpec((1, d), lambda t: (0, 0)),
            pl.BlockSpec((d, d), lambda t: (0, 0)),
            pl.BlockSpec((1, d), lambda t: (0, 0)),
            pl.BlockSpec((d, _E), lambda t: (0, 0)),
            pl.BlockSpec((1, _E), lambda t: (0, 0)),
        ],
        out_specs=[
            pl.BlockSpec((_TM, _E), lambda t: (t, 0)),
            pl.BlockSpec((_TM, 1), lambda t: (t, 0)),
            pl.BlockSpec((_TM, 1), lambda t: (t, 0)),
            pl.BlockSpec((_TM, 1), lambda t: (t, 0)),
            pl.BlockSpec((_TM, 1), lambda t: (t, 0)),
            pl.BlockSpec((_TM, 1), lambda t: (t, 0)),
            pl.BlockSpec((_TM, 1), lambda t: (t, 0)),
            pl.BlockSpec((1, _E), lambda t: (0, 0)),
            pl.BlockSpec((1, _E), lambda t: (0, 0)),
            pl.BlockSpec((1, 48), lambda t: (0, 0)),
        ],
        out_shape=[
            jax.ShapeDtypeStruct((N, _E), jnp.float32),
            jax.ShapeDtypeStruct((N, 1), jnp.int32),
            jax.ShapeDtypeStruct((N, 1), jnp.int32),
            jax.ShapeDtypeStruct((N, 1), jnp.float32),
            jax.ShapeDtypeStruct((N, 1), jnp.float32),
            jax.ShapeDtypeStruct((N, 1), jnp.int32),
            jax.ShapeDtypeStruct((N, 1), jnp.int32),
            jax.ShapeDtypeStruct((1, _E), jnp.float32),
            jax.ShapeDtypeStruct((1, _E), jnp.int32),
            jax.ShapeDtypeStruct((1, 48), jnp.int32),
        ],
        scratch_shapes=[pltpu.VMEM((1, _E), jnp.float32)],
    )(q, Wv, bv.reshape(1, d), Wo, bo.reshape(1, d),
      gate_W, gate_b.reshape(1, _E))

    counts16 = jnp.pad(counts.reshape(_E), (0, 8))
    xbi = lax.bitcast_convert_type(
        xf.astype(jnp.bfloat16).reshape(N, d // 2, 2), jnp.int32)

    mesh = plsc.VectorSubcoreMesh(
        core_axis_name="c", subcore_axis_name="s",
        num_cores=2, num_subcores=16)

    xs, ws, s1, s2 = pl.kernel(
        _dispatch_kernel,
        mesh=mesh,
        compiler_params=pltpu.CompilerParams(needs_layout_passes=False),
        out_type=[
            jax.ShapeDtypeStruct((_P, _EMB // 2), jnp.int32),
            jax.ShapeDtypeStruct((_P,), jnp.float32),
            jax.ShapeDtypeStruct((_NW, _TW), jnp.int32),
            jax.ShapeDtypeStruct((_NW, _TW), jnp.int32),
        ],
        scratch_types=[
            pltpu.VMEM((16,), jnp.int32),
            pltpu.VMEM((16,), jnp.int32),
            pltpu.VMEM((2, _TW), jnp.int32),
            pltpu.VMEM((2, _TW), jnp.int32),
            pltpu.VMEM((2, _TW), jnp.float32),
            pltpu.VMEM((2, _TW), jnp.int32),
            pltpu.VMEM((_TW, _EMB // 2), jnp.int32),
            pltpu.SemaphoreType.DMA,
        ],
    )(i1.reshape(N), i2.reshape(N), r1.reshape(N), r2.reshape(N),
      counts16, xbi, w1.reshape(N), w2.reshape(N))

    xs2 = lax.bitcast_convert_type(xs, jnp.bfloat16).reshape(_P, d)
    ws3 = ws.reshape(_NT, _TB, 1)
    eid = eid_row.reshape(48)
    w1b = W1.astype(jnp.bfloat16)
    w2b = W2.astype(jnp.bfloat16)
    b1r = b1.reshape(_E, 1, _DFF)
    b2r = b2.reshape(_E, 1, d)

    ys = pl.pallas_call(
        _expert_kernel,
        grid_spec=pltpu.PrefetchScalarGridSpec(
            num_scalar_prefetch=1,
            grid=(_NT,),
            in_specs=[
                pl.BlockSpec((_TB, d), lambda t, eid: (t, 0)),
                pl.BlockSpec((1, d, _DFF), lambda t, eid: (eid[t], 0, 0)),
                pl.BlockSpec((1, 1, _DFF), lambda t, eid: (eid[t], 0, 0)),
                pl.BlockSpec((1, _DFF, d), lambda t, eid: (eid[t], 0, 0)),
                pl.BlockSpec((1, 1, d), lambda t, eid: (eid[t], 0, 0)),
                pl.BlockSpec((1, _TB, 1), lambda t, eid: (t, 0, 0)),
            ],
            out_specs=pl.BlockSpec((_TB, d), lambda t, eid: (t, 0)),
        ),
        out_shape=jax.ShapeDtypeStruct((_P, d), jnp.bfloat16),
    )(eid, xs2, w1b, b1r, w2b, b2r, ws3)

    ysi = lax.bitcast_convert_type(
        ys.reshape(_P, d // 2, 2), jnp.int32)

    yi = pl.kernel(
        _combine_kernel,
        mesh=mesh,
        compiler_params=pltpu.CompilerParams(needs_layout_passes=False),
        out_type=jax.ShapeDtypeStruct((N, d // 2), jnp.int32),
        scratch_types=[
            pltpu.VMEM((2, _TW), jnp.int32),
            pltpu.VMEM((2, 16, _EMB // 2), jnp.int32),
            pltpu.VMEM((2, 16, _EMB // 2), jnp.int32),
            pltpu.VMEM((16, _EMB // 2), jnp.int32),
            pltpu.SemaphoreType.DMA,
            pltpu.SemaphoreType.DMA,
        ],
    )(ysi, s1, s2)

    y = lax.bitcast_convert_type(yi, jnp.bfloat16).reshape(N, d)
    y = y.astype(jnp.float32)

    importance = imp[0]
    importance_loss = _W_IMPORTANCE * (
        jnp.std(importance, ddof=1) / jnp.mean(importance)) ** 2
    return y.reshape(x_shape), gate_prob, importance_loss


# speedup vs baseline: 2.1601x; 2.1601x over previous
import jax
import jax.numpy as jnp
from jax import lax
from jax.experimental import pallas as pl
from jax.experimental.pallas import tpu as pltpu

_EMB = 1024
_DFF = 2048
_E = 8
_W_IMPORTANCE = 0.01
_TM = 1024
_TD = 1024


def _gate_kernel(q_ref, wv_ref, bv_ref, wo_ref, bo_ref, gw_ref, gb_ref,
                 prob_ref, ct_ref, imp_ref):
    t = pl.program_id(0)
    v = jnp.dot(q_ref[...], wv_ref[...], preferred_element_type=jnp.float32)
    v = v + bv_ref[...]
    att = jnp.dot(v, wo_ref[...], preferred_element_type=jnp.float32)
    att = att + bo_ref[...]
    logits = jnp.dot(att, gw_ref[...], preferred_element_type=jnp.float32)
    logits = logits + gb_ref[...]
    lmax = jnp.max(logits, axis=1, keepdims=True)
    ex = jnp.exp(logits - lmax)
    p = ex / jnp.sum(ex, axis=1, keepdims=True)
    prob_ref[...] = p

    iota = lax.broadcasted_iota(jnp.int32, p.shape, 1)
    m1 = jnp.max(p, axis=1, keepdims=True)
    i1 = jnp.min(jnp.where(p == m1, iota, _E), axis=1, keepdims=True)
    oh1 = iota == i1
    pm = jnp.where(oh1, -jnp.inf, p)
    m2 = jnp.max(pm, axis=1, keepdims=True)
    i2 = jnp.min(jnp.where(pm == m2, iota, _E), axis=1, keepdims=True)
    oh2 = iota == i2
    e21 = jnp.exp(m2 - m1)
    w1 = 1.0 / (1.0 + e21)
    w2 = e21 / (1.0 + e21)
    ct_ref[...] = jnp.where(oh1, w1, 0.0) + jnp.where(oh2, w2, 0.0)

    @pl.when(t == 0)
    def _():
        imp_ref[...] = jnp.zeros_like(imp_ref)

    imp_ref[...] += jnp.sum(p, axis=0, keepdims=True)


def _moe_kernel(x_ref, w1_ref, b1_ref, w2_ref, b2_ref, c_ref, y_ref):
    e = pl.program_id(1)
    h = jnp.dot(x_ref[...], w1_ref[0], preferred_element_type=jnp.float32)
    h = jnp.maximum(h + b1_ref[0], 0.0).astype(jnp.bfloat16)
    part = jnp.dot(h, w2_ref[0], preferred_element_type=jnp.float32)
    cb = c_ref[0]

    @pl.when(e == 0)
    def _():
        y_ref[...] = jnp.zeros_like(y_ref)

    y_ref[...] += (part + b2_ref[0]) * cb


def kernel(x, q, Wq, bq, Wk, bk, Wv, bv, Wo, bo, gate_W, gate_b, W1, b1, W2, b2):
    x_shape = x.shape
    xf = x.reshape(-1, x_shape[-1])
    N, d = xf.shape
    T = N // _TM

    gate_prob, c, imp = pl.pallas_call(
        _gate_kernel,
        grid=(T,),
        in_specs=[
            pl.BlockSpec((_TM, d), lambda t: (t, 0)),
            pl.BlockSpec((d, d), lambda t: (0, 0)),
            pl.BlockSpec((1, d), lambda t: (0, 0)),
            pl.BlockSpec((d, d), lambda t: (0, 0)),
            pl.BlockSpec((1, d), lambda t: (0, 0)),
            pl.BlockSpec((d, _E), lambda t: (0, 0)),
            pl.BlockSpec((1, _E), lambda t: (0, 0)),
        ],
        out_specs=[
            pl.BlockSpec((_TM, _E), lambda t: (t, 0)),
            pl.BlockSpec((_TM, _E), lambda t: (t, 0)),
            pl.BlockSpec((1, _E), lambda t: (0, 0)),
        ],
        out_shape=[
            jax.ShapeDtypeStruct((N, _E), jnp.float32),
            jax.ShapeDtypeStruct((N, _E), jnp.float32),
            jax.ShapeDtypeStruct((1, _E), jnp.float32),
        ],
    )(q, Wv, bv.reshape(1, d), Wo, bo.reshape(1, d),
      gate_W, gate_b.reshape(1, _E))

    cT = c.T.reshape(_E, N, 1)
    xb = xf.astype(jnp.bfloat16)
    w1b = W1.astype(jnp.bfloat16)
    w2b = W2.astype(jnp.bfloat16)
    b1r = b1.reshape(_E, 1, _DFF)
    b2r = b2.reshape(_E, 1, d)

    y = pl.pallas_call(
        _moe_kernel,
        grid=(N // _TD, _E),
        in_specs=[
            pl.BlockSpec((_TD, d), lambda i, e: (i, 0)),
            pl.BlockSpec((1, d, _DFF), lambda i, e: (e, 0, 0)),
            pl.BlockSpec((1, 1, _DFF), lambda i, e: (e, 0, 0)),
            pl.BlockSpec((1, _DFF, d), lambda i, e: (e, 0, 0)),
            pl.BlockSpec((1, 1, d), lambda i, e: (e, 0, 0)),
            pl.BlockSpec((1, _TD, 1), lambda i, e: (e, i, 0)),
        ],
        out_specs=pl.BlockSpec((_TD, d), lambda i, e: (i, 0)),
        out_shape=jax.ShapeDtypeStruct((N, d), jnp.float32),
    )(xb, w1b, b1r, w2b, b2r, cT)

    importance = imp[0]
    importance_loss = _W_IMPORTANCE * (
        jnp.std(importance, ddof=1) / jnp.mean(importance)) ** 2
    return y.reshape(x_shape), gate_prob, importance_loss
